# baseline (speedup 1.0000x reference)
import functools

import jax
import jax.numpy as jnp
from jax import lax
from jax.experimental import pallas as pl
from jax.experimental.pallas import tpu as pltpu
from jax.experimental.pallas import tpu_sc as plsc

_B, _S = 4096, 50
_D = 64
_N = _B * _S
_NW = 32
_BPW = _N // _NW
_C = 128
_NCHUNK = _BPW // _C
_L = 16


def _make_gather():
    mesh = plsc.VectorSubcoreMesh(core_axis_name="c", subcore_axis_name="s")

    @functools.partial(
        pl.kernel,
        mesh=mesh,
        out_type=jax.ShapeDtypeStruct((_N, _D), jnp.float32),
        scratch_types=[
            pltpu.VMEM((_BPW,), jnp.int32),
            pltpu.VMEM((_BPW,), jnp.int32),
            pltpu.VMEM((2, _C, 2 * _D), jnp.float32),
            pltpu.VMEM((2, _C, _D), jnp.float32),
            pltpu.SemaphoreType.DMA,
            pltpu.SemaphoreType.DMA,
        ],
        compiler_params=pltpu.CompilerParams(
            use_tc_tiling_on_sc=True, needs_layout_passes=False),
    )
    def gather_kernel(rows_hbm, halves_hbm, table_hbm, out_hbm,
                      rows_v, halves_v, gbuf, pbuf, gsem, wsem):
        wid = lax.axis_index("s") * 2 + lax.axis_index("c")
        base = wid * _BPW
        pltpu.sync_copy(rows_hbm.at[pl.ds(base, _BPW)], rows_v)
        pltpu.sync_copy(halves_hbm.at[pl.ds(base, _BPW)], halves_v)

        def start_gather(k, b):
            pltpu.async_copy(
                table_hbm.at[rows_v.at[pl.ds(k * _C, _C)]], gbuf.at[b], gsem)

        def wait_gather(b):
            pltpu.make_async_copy(
                table_hbm.at[rows_v.at[pl.ds(0, _C)]], gbuf.at[b], gsem).wait()

        def start_write(k, b):
            pltpu.async_copy(
                pbuf.at[b], out_hbm.at[pl.ds(base + k * _C, _C)], wsem)

        def wait_write(b):
            pltpu.make_async_copy(
                pbuf.at[b], out_hbm.at[pl.ds(base, _C)], wsem).wait()

        def select(k, b):
            bvec = jnp.full((_L,), 0, jnp.int32) + b

            def group(g, _):
                jv = g * _L + lax.iota(jnp.int32, _L)
                h = plsc.load_gather(halves_v, [k * _C + jv])
                coff = h * _D
                for c in range(_D):
                    v = plsc.load_gather(gbuf, [bvec, jv, coff + c])
                    plsc.store_scatter(
                        pbuf, [bvec, jv, jnp.full((_L,), c, jnp.int32)], v)
                return 0

            lax.fori_loop(0, _C // _L, group, 0)

        start_gather(0, 0)
        start_gather(1, 1)

        def step(k, _):
            b = lax.rem(k, 2)
            wait_gather(b)

            @pl.when(k >= 2)
            def _():
                wait_write(b)

            select(k, b)
            start_write(k, b)

            @pl.when(k + 2 < _NCHUNK)
            def _():
                start_gather(k + 2, b)

            return 0

        lax.fori_loop(0, _NCHUNK, step, 0)
        wait_write(0)
        wait_write(1)

    return gather_kernel


_gather = _make_gather()


@jax.jit
def kernel(token_ids, weight):
    flat_ids = token_ids.reshape(_N).astype(jnp.int32)
    rows = flat_ids >> 1
    halves = flat_ids & 1
    w128 = weight.reshape(500000, 128)
    out = _gather(rows, halves, w128)
    return out.reshape(_B, _S, _D)


# speedup vs baseline: 1.1984x; 1.1984x over previous
import functools

import jax
import jax.numpy as jnp
from jax import lax
from jax.experimental import pallas as pl
from jax.experimental.pallas import tpu as pltpu
from jax.experimental.pallas import tpu_sc as plsc

_B, _S = 4096, 50
_D = 64
_V = 1000000
_N = _B * _S
_NW = 32
_L = 16

_TB = 2048

_BPW = _N // _NW
_C = 200
_NCHUNK = _BPW // _C
_BPC = _C // _S
_NG = -(-_C // _L)


_NTB = -(-_V // _TB)
_WR = _NTB * (_TB // 2)


def _make_transpose():
    def body(i_ref, o_ref):
        x = i_ref[...]
        h = _TB // 2
        o_ref[:, 0:_D] = x[:, 0:h].T
        o_ref[:, _D:2 * _D] = x[:, h:_TB].T

    return pl.pallas_call(
        body,
        grid=(_NTB,),
        in_specs=[pl.BlockSpec((_D, _TB), lambda i: (0, i))],
        out_specs=pl.BlockSpec((_TB // 2, 2 * _D), lambda i: (i, 0)),
        out_shape=jax.ShapeDtypeStruct((_WR, 2 * _D), jnp.float32),
    )


def _make_gather():
    mesh = plsc.VectorSubcoreMesh(core_axis_name="c", subcore_axis_name="s")

    @functools.partial(
        pl.kernel,
        mesh=mesh,
        out_type=jax.ShapeDtypeStruct((_B, _S, _D), jnp.float32),
        scratch_types=[
            pltpu.VMEM((_BPW,), jnp.int32),
            pltpu.VMEM((_BPW,), jnp.int32),
            pltpu.VMEM((_C, 2 * _D), jnp.float32),
            pltpu.VMEM((_C, 2 * _D), jnp.float32),
            [pltpu.VMEM((_S, _D), jnp.float32) for _ in range(_BPC)],
            [pltpu.VMEM((_S, _D), jnp.float32) for _ in range(_BPC)],
            pltpu.SemaphoreType.DMA,
            pltpu.SemaphoreType.DMA,
            pltpu.SemaphoreType.DMA,
            pltpu.SemaphoreType.DMA,
        ],
        compiler_params=pltpu.CompilerParams(
            use_tc_tiling_on_sc=True, needs_layout_passes=False),
    )
    def gather_kernel(rows_hbm, halves_hbm, table_hbm, out_hbm,
                      rows_v, halves_v, gbuf_a, gbuf_b, pbuf_a, pbuf_b,
                      gsem_a, gsem_b, wsem_a, wsem_b):
        wid = lax.axis_index("s") * 2 + lax.axis_index("c")
        base = wid * _BPW
        bbase = wid * (_BPW // _S)
        pltpu.sync_copy(rows_hbm.at[pl.ds(base, _BPW)], rows_v)
        pltpu.sync_copy(halves_hbm.at[pl.ds(base, _BPW)], halves_v)

        def start_gather(k, gbuf, gsem):
            pltpu.async_copy(
                table_hbm.at[rows_v.at[pl.ds(k * _C, _C)]], gbuf, gsem)

        def wait_gather(gbuf, gsem):
            pltpu.make_async_copy(
                table_hbm.at[rows_v.at[pl.ds(0, _C)]], gbuf, gsem).wait()

        def start_writes(k, pbufs, wsem):
            for i in range(_BPC):
                pltpu.async_copy(
                    pbufs[i], out_hbm.at[bbase + k * _BPC + i], wsem)

        def wait_writes(pbufs, wsem):
            for i in range(_BPC):
                pltpu.make_async_copy(pbufs[i], out_hbm.at[0], wsem).wait()

        def select(k, gbuf, pbufs):
            iota = lax.iota(jnp.int32, _L)
            for i in range(_BPC):
                groups = []
                for r0 in range(0, _S, _L):
                    nlane = min(_L, _S - r0)
                    rv = r0 + iota
                    msk = iota < nlane
                    jv = i * _S + rv
                    h = plsc.load_gather(halves_v, [k * _C + jv], mask=msk)
                    groups.append((rv, jv, h * _D, msk))

                def col(c, _):
                    for rv, jv, coff, msk in groups:
                        v = plsc.load_gather(gbuf, [jv, coff + c], mask=msk)
                        plsc.store_scatter(
                            pbufs[i], [rv, jnp.full((_L,), c, jnp.int32)],
                            v, mask=msk)
                    return 0

                lax.fori_loop(0, _D, col, 0)

        start_gather(0, gbuf_a, gsem_a)
        start_gather(1, gbuf_b, gsem_b)

        def pair(m, _):
            ka = 2 * m
            kb = 2 * m + 1

            wait_gather(gbuf_a, gsem_a)

            @pl.when(m >= 1)
            def _():
                wait_writes(pbuf_a, wsem_a)

            select(ka, gbuf_a, pbuf_a)
            start_writes(ka, pbuf_a, wsem_a)

            @pl.when(ka + 2 < _NCHUNK)
            def _():
                start_gather(ka + 2, gbuf_a, gsem_a)

            wait_gather(gbuf_b, gsem_b)

            @pl.when(m >= 1)
            def _():
                wait_writes(pbuf_b, wsem_b)

            select(kb, gbuf_b, pbuf_b)
            start_writes(kb, pbuf_b, wsem_b)

            @pl.when(kb + 2 < _NCHUNK)
            def _():
                start_gather(kb + 2, gbuf_b, gsem_b)

            return 0

        lax.fori_loop(0, _NCHUNK // 2, pair, 0)
        wait_writes(pbuf_a, wsem_a)
        wait_writes(pbuf_b, wsem_b)

    return gather_kernel


_transpose = _make_transpose()
_gather = _make_gather()


@jax.jit
def kernel(token_ids, weight):
    flat_ids = token_ids.reshape(_N).astype(jnp.int32)
    blk = flat_ids >> 11
    off = flat_ids & 2047
    rows = (blk << 10) | (off & 1023)
    halves = off >> 10
    table128 = _transpose(weight.T)
    return _gather(rows, halves, table128)


# speedup vs baseline: 1.7544x; 1.4639x over previous
import functools

import jax
import jax.numpy as jnp
from jax import lax
from jax.experimental import pallas as pl
from jax.experimental.pallas import tpu as pltpu
from jax.experimental.pallas import tpu_sc as plsc

_B, _S = 4096, 50
_D = 64
_V = 1000000
_N = _B * _S
_NW = 32
_L = 16

_TB = 2048

_BPW = _N // _NW
_C = 200
_NCHUNK = _BPW // _C
_BPC = _C // _S
_NG = -(-_C // _L)


_NTB = -(-_V // _TB)
_WR = _NTB * (_TB // 2)


def _make_transpose():
    def body(i_ref, o_ref):
        x = i_ref[...]
        h = _TB // 2
        o_ref[:, 0:_D] = x[:, 0:h].T
        o_ref[:, _D:2 * _D] = x[:, h:_TB].T

    return pl.pallas_call(
        body,
        grid=(_NTB,),
        in_specs=[pl.BlockSpec((_D, _TB), lambda i: (0, i))],
        out_specs=pl.BlockSpec((_TB // 2, 2 * _D), lambda i: (i, 0)),
        out_shape=jax.ShapeDtypeStruct((_WR, 2 * _D), jnp.float32),
    )


def _make_gather():
    mesh = plsc.VectorSubcoreMesh(core_axis_name="c", subcore_axis_name="s")

    @functools.partial(
        pl.kernel,
        mesh=mesh,
        out_type=jax.ShapeDtypeStruct((_B, _S, _D), jnp.float32),
        scratch_types=[
            pltpu.VMEM((_BPW,), jnp.int32),
            pltpu.VMEM((_BPW,), jnp.int32),
            pltpu.VMEM((_C, 2 * _D), jnp.float32),
            pltpu.VMEM((_C, 2 * _D), jnp.float32),
            [pltpu.VMEM((_S, _D), jnp.float32) for _ in range(_BPC)],
            [pltpu.VMEM((_S, _D), jnp.float32) for _ in range(_BPC)],
            pltpu.SemaphoreType.DMA,
            pltpu.SemaphoreType.DMA,
            pltpu.SemaphoreType.DMA,
            pltpu.SemaphoreType.DMA,
        ],
        compiler_params=pltpu.CompilerParams(
            use_tc_tiling_on_sc=True, needs_layout_passes=False),
    )
    def gather_kernel(rows_hbm, halves_hbm, table_hbm, out_hbm,
                      rows_v, halves_v, gbuf_a, gbuf_b, pbuf_a, pbuf_b,
                      gsem_a, gsem_b, wsem_a, wsem_b):
        wid = lax.axis_index("s") * 2 + lax.axis_index("c")
        base = wid * _BPW
        bbase = wid * (_BPW // _S)
        pltpu.sync_copy(rows_hbm.at[pl.ds(base, _BPW)], rows_v)
        pltpu.sync_copy(halves_hbm.at[pl.ds(base, _BPW)], halves_v)

        def start_gather(k, gbuf, gsem):
            pltpu.async_copy(
                table_hbm.at[rows_v.at[pl.ds(k * _C, _C)]], gbuf, gsem)

        def wait_gather(gbuf, gsem):
            pltpu.make_async_copy(
                table_hbm.at[rows_v.at[pl.ds(0, _C)]], gbuf, gsem).wait()

        def start_writes(k, pbufs, wsem):
            for i in range(_BPC):
                pltpu.async_copy(
                    pbufs[i], out_hbm.at[bbase + k * _BPC + i], wsem)

        def wait_writes(pbufs, wsem):
            for i in range(_BPC):
                pltpu.make_async_copy(pbufs[i], out_hbm.at[0], wsem).wait()

        def select(k, gbuf, pbufs):
            iota = lax.iota(jnp.int32, _L)
            for i in range(_BPC):
                pbuf = pbufs[i]

                def row(r, _):
                    j = i * _S + r
                    jsplat = jnp.full((_L,), 0, jnp.int32) + j
                    h = plsc.load_gather(halves_v, [k * _C + jsplat])
                    coff = h * _D + iota
                    rsplat = jnp.full((_L,), 0, jnp.int32) + r
                    for cg in range(_D // _L):
                        v = plsc.load_gather(gbuf, [jsplat, coff + cg * _L])
                        plsc.store_scatter(
                            pbuf, [rsplat, iota + cg * _L], v)
                    return 0

                lax.fori_loop(0, _S, row, 0)

        start_gather(0, gbuf_a, gsem_a)
        start_gather(1, gbuf_b, gsem_b)

        def pair(m, _):
            ka = 2 * m
            kb = 2 * m + 1

            wait_gather(gbuf_a, gsem_a)

            @pl.when(m >= 1)
            def _():
                wait_writes(pbuf_a, wsem_a)

            select(ka, gbuf_a, pbuf_a)
            start_writes(ka, pbuf_a, wsem_a)

            @pl.when(ka + 2 < _NCHUNK)
            def _():
                start_gather(ka + 2, gbuf_a, gsem_a)

            wait_gather(gbuf_b, gsem_b)

            @pl.when(m >= 1)
            def _():
                wait_writes(pbuf_b, wsem_b)

            select(kb, gbuf_b, pbuf_b)
            start_writes(kb, pbuf_b, wsem_b)

            @pl.when(kb + 2 < _NCHUNK)
            def _():
                start_gather(kb + 2, gbuf_b, gsem_b)

            return 0

        lax.fori_loop(0, _NCHUNK // 2, pair, 0)
        wait_writes(pbuf_a, wsem_a)
        wait_writes(pbuf_b, wsem_b)

    return gather_kernel


_transpose = _make_transpose()
_gather = _make_gather()


@jax.jit
def kernel(token_ids, weight):
    flat_ids = token_ids.reshape(_N).astype(jnp.int32)
    blk = flat_ids >> 11
    off = flat_ids & 2047
    rows = (blk << 10) | (off & 1023)
    halves = off >> 10
    table128 = _transpose(weight.T)
    return _gather(rows, halves, table128)


# speedup vs baseline: 2.1168x; 1.2066x over previous
import functools

import jax
import jax.numpy as jnp
from jax import lax
from jax.experimental import pallas as pl
from jax.experimental.pallas import tpu as pltpu
from jax.experimental.pallas import tpu_sc as plsc

_B, _S = 4096, 50
_D = 64
_V = 1000000
_N = _B * _S
_NW = 32
_L = 16

_TB = 4096

_BPW = _N // _NW
_C = 200
_NCHUNK = _BPW // _C
_BPC = _C // _S
_NG = -(-_C // _L)


_NTB = -(-_V // _TB)
_WR = _NTB * (_TB // 2)


def _make_transpose():
    def body(i_ref, o_ref):
        x = i_ref[...]
        h = _TB // 2
        o_ref[:, 0:_D] = x[:, 0:h].T
        o_ref[:, _D:2 * _D] = x[:, h:_TB].T

    return pl.pallas_call(
        body,
        grid=(_NTB,),
        in_specs=[pl.BlockSpec((_D, _TB), lambda i: (0, i))],
        out_specs=pl.BlockSpec((_TB // 2, 2 * _D), lambda i: (i, 0)),
        out_shape=jax.ShapeDtypeStruct((_WR, 2 * _D), jnp.float32),
    )


def _make_gather():
    mesh = plsc.VectorSubcoreMesh(core_axis_name="c", subcore_axis_name="s")

    @functools.partial(
        pl.kernel,
        mesh=mesh,
        out_type=jax.ShapeDtypeStruct((_B, _S, _D), jnp.float32),
        scratch_types=[
            pltpu.VMEM((_BPW,), jnp.int32),
            pltpu.VMEM((_BPW,), jnp.int32),
            pltpu.VMEM((_C, 2 * _D), jnp.float32),
            pltpu.VMEM((_C, 2 * _D), jnp.float32),
            [pltpu.VMEM((_S, _D), jnp.float32) for _ in range(_BPC)],
            [pltpu.VMEM((_S, _D), jnp.float32) for _ in range(_BPC)],
            pltpu.SemaphoreType.DMA,
            pltpu.SemaphoreType.DMA,
            pltpu.SemaphoreType.DMA,
            pltpu.SemaphoreType.DMA,
        ],
        compiler_params=pltpu.CompilerParams(
            use_tc_tiling_on_sc=True, needs_layout_passes=False),
    )
    def gather_kernel(rows_hbm, halves_hbm, table_hbm, out_hbm,
                      rows_v, halves_v, gbuf_a, gbuf_b, pbuf_a, pbuf_b,
                      gsem_a, gsem_b, wsem_a, wsem_b):
        wid = lax.axis_index("s") * 2 + lax.axis_index("c")
        base = wid * _BPW
        bbase = wid * (_BPW // _S)
        pltpu.sync_copy(rows_hbm.at[pl.ds(base, _BPW)], rows_v)
        pltpu.sync_copy(halves_hbm.at[pl.ds(base, _BPW)], halves_v)

        def start_gather(k, gbuf, gsem):
            pltpu.async_copy(
                table_hbm.at[rows_v.at[pl.ds(k * _C, _C)]], gbuf, gsem)

        def wait_gather(gbuf, gsem):
            pltpu.make_async_copy(
                table_hbm.at[rows_v.at[pl.ds(0, _C)]], gbuf, gsem).wait()

        def start_writes(k, pbufs, wsem):
            for i in range(_BPC):
                pltpu.async_copy(
                    pbufs[i], out_hbm.at[bbase + k * _BPC + i], wsem)

        def wait_writes(pbufs, wsem):
            for i in range(_BPC):
                pltpu.make_async_copy(pbufs[i], out_hbm.at[0], wsem).wait()

        def select(k, gbuf, pbufs):
            iota = lax.iota(jnp.int32, _L)
            for i in range(_BPC):
                pbuf = pbufs[i]

                def row(r, _):
                    j = i * _S + r
                    jsplat = jnp.full((_L,), 0, jnp.int32) + j
                    h = plsc.load_gather(halves_v, [k * _C + jsplat])
                    coff = h * _D + iota
                    rsplat = jnp.full((_L,), 0, jnp.int32) + r
                    for cg in range(_D // _L):
                        v = plsc.load_gather(gbuf, [jsplat, coff + cg * _L])
                        plsc.store_scatter(
                            pbuf, [rsplat, iota + cg * _L], v)
                    return 0

                lax.fori_loop(0, _S, row, 0)

        start_gather(0, gbuf_a, gsem_a)
        start_gather(1, gbuf_b, gsem_b)

        def pair(m, _):
            ka = 2 * m
            kb = 2 * m + 1

            wait_gather(gbuf_a, gsem_a)

            @pl.when(m >= 1)
            def _():
                wait_writes(pbuf_a, wsem_a)

            select(ka, gbuf_a, pbuf_a)
            start_writes(ka, pbuf_a, wsem_a)

            @pl.when(ka + 2 < _NCHUNK)
            def _():
                start_gather(ka + 2, gbuf_a, gsem_a)

            wait_gather(gbuf_b, gsem_b)

            @pl.when(m >= 1)
            def _():
                wait_writes(pbuf_b, wsem_b)

            select(kb, gbuf_b, pbuf_b)
            start_writes(kb, pbuf_b, wsem_b)

            @pl.when(kb + 2 < _NCHUNK)
            def _():
                start_gather(kb + 2, gbuf_b, gsem_b)

            return 0

        lax.fori_loop(0, _NCHUNK // 2, pair, 0)
        wait_writes(pbuf_a, wsem_a)
        wait_writes(pbuf_b, wsem_b)

    return gather_kernel


_transpose = _make_transpose()
_gather = _make_gather()


@jax.jit
def kernel(token_ids, weight):
    flat_ids = token_ids.reshape(_N).astype(jnp.int32)
    shift = _TB.bit_length() - 1
    blk = flat_ids >> shift
    off = flat_ids & (_TB - 1)
    rows = (blk << (shift - 1)) | (off & (_TB // 2 - 1))
    halves = off >> (shift - 1)
    table128 = _transpose(weight.T)
    return _gather(rows, halves, table128)


# speedup vs baseline: 2.3918x; 1.1299x over previous
import functools

import jax
import jax.numpy as jnp
from jax import lax
from jax.experimental import pallas as pl
from jax.experimental.pallas import tpu as pltpu
from jax.experimental.pallas import tpu_sc as plsc

_B, _S = 4096, 50
_D = 64
_V = 1000000
_N = _B * _S
_NW = 32
_L = 16

_TB = 8192

_BPW = _N // _NW
_C = 200
_NCHUNK = _BPW // _C
_BPC = _C // _S
_NG = -(-_C // _L)


_NTB = -(-_V // _TB)
_WR = _NTB * (_TB // 2)


def _make_transpose():
    def body(i_ref, o_ref):
        x = i_ref[...]
        h = _TB // 2
        o_ref[:, 0:_D] = x[:, 0:h].T
        o_ref[:, _D:2 * _D] = x[:, h:_TB].T

    return pl.pallas_call(
        body,
        grid=(_NTB,),
        in_specs=[pl.BlockSpec((_D, _TB), lambda i: (0, i))],
        out_specs=pl.BlockSpec((_TB // 2, 2 * _D), lambda i: (i, 0)),
        out_shape=jax.ShapeDtypeStruct((_WR, 2 * _D), jnp.float32),
    )


def _make_gather():
    mesh = plsc.VectorSubcoreMesh(core_axis_name="c", subcore_axis_name="s")

    @functools.partial(
        pl.kernel,
        mesh=mesh,
        out_type=jax.ShapeDtypeStruct((_B, _S, _D), jnp.float32),
        scratch_types=[
            pltpu.VMEM((_BPW,), jnp.int32),
            pltpu.VMEM((_BPW,), jnp.int32),
            pltpu.VMEM((_C, 2 * _D), jnp.float32),
            pltpu.VMEM((_C, 2 * _D), jnp.float32),
            [pltpu.VMEM((_S, _D), jnp.float32) for _ in range(_BPC)],
            [pltpu.VMEM((_S, _D), jnp.float32) for _ in range(_BPC)],
            pltpu.SemaphoreType.DMA,
            pltpu.SemaphoreType.DMA,
            pltpu.SemaphoreType.DMA,
            pltpu.SemaphoreType.DMA,
        ],
        compiler_params=pltpu.CompilerParams(
            use_tc_tiling_on_sc=True, needs_layout_passes=False),
    )
    def gather_kernel(rows_hbm, halves_hbm, table_hbm, out_hbm,
                      rows_v, halves_v, gbuf_a, gbuf_b, pbuf_a, pbuf_b,
                      gsem_a, gsem_b, wsem_a, wsem_b):
        wid = lax.axis_index("s") * 2 + lax.axis_index("c")
        base = wid * _BPW
        bbase = wid * (_BPW // _S)
        pltpu.sync_copy(rows_hbm.at[pl.ds(base, _BPW)], rows_v)
        pltpu.sync_copy(halves_hbm.at[pl.ds(base, _BPW)], halves_v)

        def start_gather(k, gbuf, gsem):
            pltpu.async_copy(
                table_hbm.at[rows_v.at[pl.ds(k * _C, _C)]], gbuf, gsem)

        def wait_gather(gbuf, gsem):
            pltpu.make_async_copy(
                table_hbm.at[rows_v.at[pl.ds(0, _C)]], gbuf, gsem).wait()

        def start_writes(k, pbufs, wsem):
            for i in range(_BPC):
                pltpu.async_copy(
                    pbufs[i], out_hbm.at[bbase + k * _BPC + i], wsem)

        def wait_writes(pbufs, wsem):
            for i in range(_BPC):
                pltpu.make_async_copy(pbufs[i], out_hbm.at[0], wsem).wait()

        def select(k, gbuf, pbufs):
            iota = lax.iota(jnp.int32, _L)
            for i in range(_BPC):
                pbuf = pbufs[i]

                def row(r, _):
                    j = i * _S + r
                    jsplat = jnp.full((_L,), 0, jnp.int32) + j
                    h = plsc.load_gather(halves_v, [k * _C + jsplat])
                    coff = h * _D + iota
                    rsplat = jnp.full((_L,), 0, jnp.int32) + r
                    for cg in range(_D // _L):
                        v = plsc.load_gather(gbuf, [jsplat, coff + cg * _L])
                        plsc.store_scatter(
                            pbuf, [rsplat, iota + cg * _L], v)
                    return 0

                lax.fori_loop(0, _S, row, 0)

        start_gather(0, gbuf_a, gsem_a)
        start_gather(1, gbuf_b, gsem_b)

        def pair(m, _):
            ka = 2 * m
            kb = 2 * m + 1

            wait_gather(gbuf_a, gsem_a)

            @pl.when(m >= 1)
            def _():
                wait_writes(pbuf_a, wsem_a)

            select(ka, gbuf_a, pbuf_a)
            start_writes(ka, pbuf_a, wsem_a)

            @pl.when(ka + 2 < _NCHUNK)
            def _():
                start_gather(ka + 2, gbuf_a, gsem_a)

            wait_gather(gbuf_b, gsem_b)

            @pl.when(m >= 1)
            def _():
                wait_writes(pbuf_b, wsem_b)

            select(kb, gbuf_b, pbuf_b)
            start_writes(kb, pbuf_b, wsem_b)

            @pl.when(kb + 2 < _NCHUNK)
            def _():
                start_gather(kb + 2, gbuf_b, gsem_b)

            return 0

        lax.fori_loop(0, _NCHUNK // 2, pair, 0)
        wait_writes(pbuf_a, wsem_a)
        wait_writes(pbuf_b, wsem_b)

    return gather_kernel


_transpose = _make_transpose()
_gather = _make_gather()


@jax.jit
def kernel(token_ids, weight):
    flat_ids = token_ids.reshape(_N).astype(jnp.int32)
    shift = _TB.bit_length() - 1
    blk = flat_ids >> shift
    off = flat_ids & (_TB - 1)
    rows = (blk << (shift - 1)) | (off & (_TB // 2 - 1))
    halves = off >> (shift - 1)
    table128 = _transpose(weight.T)
    return _gather(rows, halves, table128)


# speedup vs baseline: 2.5526x; 1.0672x over previous
import functools

import jax
import jax.numpy as jnp
from jax import lax
from jax.experimental import pallas as pl
from jax.experimental.pallas import tpu as pltpu
from jax.experimental.pallas import tpu_sc as plsc

_B, _S = 4096, 50
_D = 64
_V = 1000000
_N = _B * _S
_NW = 32
_L = 16

_TB = 16384

_BPW = _N // _NW
_C = 200
_NCHUNK = _BPW // _C
_BPC = _C // _S
_NG = -(-_C // _L)


_NTB = -(-_V // _TB)
_WR = _NTB * (_TB // 2)


def _make_transpose():
    def body(i_ref, o_ref):
        x = i_ref[...]
        h = _TB // 2
        o_ref[:, 0:_D] = x[:, 0:h].T
        o_ref[:, _D:2 * _D] = x[:, h:_TB].T

    return pl.pallas_call(
        body,
        grid=(_NTB,),
        in_specs=[pl.BlockSpec((_D, _TB), lambda i: (0, i))],
        out_specs=pl.BlockSpec((_TB // 2, 2 * _D), lambda i: (i, 0)),
        out_shape=jax.ShapeDtypeStruct((_WR, 2 * _D), jnp.float32),
    )


def _make_gather():
    mesh = plsc.VectorSubcoreMesh(core_axis_name="c", subcore_axis_name="s")

    @functools.partial(
        pl.kernel,
        mesh=mesh,
        out_type=jax.ShapeDtypeStruct((_B, _S, _D), jnp.float32),
        scratch_types=[
            pltpu.VMEM((_BPW,), jnp.int32),
            pltpu.VMEM((_BPW,), jnp.int32),
            pltpu.VMEM((_C, 2 * _D), jnp.float32),
            pltpu.VMEM((_C, 2 * _D), jnp.float32),
            [pltpu.VMEM((_S, _D), jnp.float32) for _ in range(_BPC)],
            [pltpu.VMEM((_S, _D), jnp.float32) for _ in range(_BPC)],
            pltpu.SemaphoreType.DMA,
            pltpu.SemaphoreType.DMA,
            pltpu.SemaphoreType.DMA,
            pltpu.SemaphoreType.DMA,
        ],
        compiler_params=pltpu.CompilerParams(
            use_tc_tiling_on_sc=True, needs_layout_passes=False),
    )
    def gather_kernel(rows_hbm, halves_hbm, table_hbm, out_hbm,
                      rows_v, halves_v, gbuf_a, gbuf_b, pbuf_a, pbuf_b,
                      gsem_a, gsem_b, wsem_a, wsem_b):
        wid = lax.axis_index("s") * 2 + lax.axis_index("c")
        base = wid * _BPW
        bbase = wid * (_BPW // _S)
        pltpu.sync_copy(rows_hbm.at[pl.ds(base, _BPW)], rows_v)
        pltpu.sync_copy(halves_hbm.at[pl.ds(base, _BPW)], halves_v)

        def start_gather(k, gbuf, gsem):
            pltpu.async_copy(
                table_hbm.at[rows_v.at[pl.ds(k * _C, _C)]], gbuf, gsem)

        def wait_gather(gbuf, gsem):
            pltpu.make_async_copy(
                table_hbm.at[rows_v.at[pl.ds(0, _C)]], gbuf, gsem).wait()

        def start_writes(k, pbufs, wsem):
            for i in range(_BPC):
                pltpu.async_copy(
                    pbufs[i], out_hbm.at[bbase + k * _BPC + i], wsem)

        def wait_writes(pbufs, wsem):
            for i in range(_BPC):
                pltpu.make_async_copy(pbufs[i], out_hbm.at[0], wsem).wait()

        def select(k, gbuf, pbufs):
            iota = lax.iota(jnp.int32, _L)
            for i in range(_BPC):
                pbuf = pbufs[i]

                def row(r, _):
                    j = i * _S + r
                    jsplat = jnp.full((_L,), 0, jnp.int32) + j
                    h = plsc.load_gather(halves_v, [k * _C + jsplat])
                    coff = h * _D + iota
                    rsplat = jnp.full((_L,), 0, jnp.int32) + r
                    for cg in range(_D // _L):
                        v = plsc.load_gather(gbuf, [jsplat, coff + cg * _L])
                        plsc.store_scatter(
                            pbuf, [rsplat, iota + cg * _L], v)
                    return 0

                lax.fori_loop(0, _S, row, 0)

        start_gather(0, gbuf_a, gsem_a)
        start_gather(1, gbuf_b, gsem_b)

        def pair(m, _):
            ka = 2 * m
            kb = 2 * m + 1

            wait_gather(gbuf_a, gsem_a)

            @pl.when(m >= 1)
            def _():
                wait_writes(pbuf_a, wsem_a)

            select(ka, gbuf_a, pbuf_a)
            start_writes(ka, pbuf_a, wsem_a)

            @pl.when(ka + 2 < _NCHUNK)
            def _():
                start_gather(ka + 2, gbuf_a, gsem_a)

            wait_gather(gbuf_b, gsem_b)

            @pl.when(m >= 1)
            def _():
                wait_writes(pbuf_b, wsem_b)

            select(kb, gbuf_b, pbuf_b)
            start_writes(kb, pbuf_b, wsem_b)

            @pl.when(kb + 2 < _NCHUNK)
            def _():
                start_gather(kb + 2, gbuf_b, gsem_b)

            return 0

        lax.fori_loop(0, _NCHUNK // 2, pair, 0)
        wait_writes(pbuf_a, wsem_a)
        wait_writes(pbuf_b, wsem_b)

    return gather_kernel


_transpose = _make_transpose()
_gather = _make_gather()


@jax.jit
def kernel(token_ids, weight):
    flat_ids = token_ids.reshape(_N).astype(jnp.int32)
    shift = _TB.bit_length() - 1
    blk = flat_ids >> shift
    off = flat_ids & (_TB - 1)
    rows = (blk << (shift - 1)) | (off & (_TB // 2 - 1))
    halves = off >> (shift - 1)
    table128 = _transpose(weight.T)
    return _gather(rows, halves, table128)


# speedup vs baseline: 2.6319x; 1.0310x over previous
import functools

import jax
import jax.numpy as jnp
from jax import lax
from jax.experimental import pallas as pl
from jax.experimental.pallas import tpu as pltpu
from jax.experimental.pallas import tpu_sc as plsc

_B, _S = 4096, 50
_D = 64
_V = 1000000
_N = _B * _S
_NW = 32
_L = 16

_TB = 32768

_BPW = _N // _NW
_C = 200
_NCHUNK = _BPW // _C
_BPC = _C // _S
_NG = -(-_C // _L)


_NTB = -(-_V // _TB)
_WR = _NTB * (_TB // 2)


def _make_transpose():
    def body(i_ref, o_ref):
        x = i_ref[...]
        h = _TB // 2
        o_ref[:, 0:_D] = x[:, 0:h].T
        o_ref[:, _D:2 * _D] = x[:, h:_TB].T

    return pl.pallas_call(
        body,
        grid=(_NTB,),
        in_specs=[pl.BlockSpec((_D, _TB), lambda i: (0, i))],
        out_specs=pl.BlockSpec((_TB // 2, 2 * _D), lambda i: (i, 0)),
        out_shape=jax.ShapeDtypeStruct((_WR, 2 * _D), jnp.float32),
    )


def _make_gather():
    mesh = plsc.VectorSubcoreMesh(core_axis_name="c", subcore_axis_name="s")

    @functools.partial(
        pl.kernel,
        mesh=mesh,
        out_type=jax.ShapeDtypeStruct((_B, _S, _D), jnp.float32),
        scratch_types=[
            pltpu.VMEM((_BPW,), jnp.int32),
            pltpu.VMEM((_BPW,), jnp.int32),
            pltpu.VMEM((_C, 2 * _D), jnp.float32),
            pltpu.VMEM((_C, 2 * _D), jnp.float32),
            [pltpu.VMEM((_S, _D), jnp.float32) for _ in range(_BPC)],
            [pltpu.VMEM((_S, _D), jnp.float32) for _ in range(_BPC)],
            pltpu.SemaphoreType.DMA,
            pltpu.SemaphoreType.DMA,
            pltpu.SemaphoreType.DMA,
            pltpu.SemaphoreType.DMA,
        ],
        compiler_params=pltpu.CompilerParams(
            use_tc_tiling_on_sc=True, needs_layout_passes=False),
    )
    def gather_kernel(rows_hbm, halves_hbm, table_hbm, out_hbm,
                      rows_v, halves_v, gbuf_a, gbuf_b, pbuf_a, pbuf_b,
                      gsem_a, gsem_b, wsem_a, wsem_b):
        wid = lax.axis_index("s") * 2 + lax.axis_index("c")
        base = wid * _BPW
        bbase = wid * (_BPW // _S)
        pltpu.sync_copy(rows_hbm.at[pl.ds(base, _BPW)], rows_v)
        pltpu.sync_copy(halves_hbm.at[pl.ds(base, _BPW)], halves_v)

        def start_gather(k, gbuf, gsem):
            pltpu.async_copy(
                table_hbm.at[rows_v.at[pl.ds(k * _C, _C)]], gbuf, gsem)

        def wait_gather(gbuf, gsem):
            pltpu.make_async_copy(
                table_hbm.at[rows_v.at[pl.ds(0, _C)]], gbuf, gsem).wait()

        def start_writes(k, pbufs, wsem):
            for i in range(_BPC):
                pltpu.async_copy(
                    pbufs[i], out_hbm.at[bbase + k * _BPC + i], wsem)

        def wait_writes(pbufs, wsem):
            for i in range(_BPC):
                pltpu.make_async_copy(pbufs[i], out_hbm.at[0], wsem).wait()

        def select(k, gbuf, pbufs):
            iota = lax.iota(jnp.int32, _L)
            for i in range(_BPC):
                pbuf = pbufs[i]

                def row(r, _):
                    j = i * _S + r
                    jsplat = jnp.full((_L,), 0, jnp.int32) + j
                    h = plsc.load_gather(halves_v, [k * _C + jsplat])
                    coff = h * _D + iota
                    rsplat = jnp.full((_L,), 0, jnp.int32) + r
                    for cg in range(_D // _L):
                        v = plsc.load_gather(gbuf, [jsplat, coff + cg * _L])
                        plsc.store_scatter(
                            pbuf, [rsplat, iota + cg * _L], v)
                    return 0

                lax.fori_loop(0, _S, row, 0)

        start_gather(0, gbuf_a, gsem_a)
        start_gather(1, gbuf_b, gsem_b)

        def pair(m, _):
            ka = 2 * m
            kb = 2 * m + 1

            wait_gather(gbuf_a, gsem_a)

            @pl.when(m >= 1)
            def _():
                wait_writes(pbuf_a, wsem_a)

            select(ka, gbuf_a, pbuf_a)
            start_writes(ka, pbuf_a, wsem_a)

            @pl.when(ka + 2 < _NCHUNK)
            def _():
                start_gather(ka + 2, gbuf_a, gsem_a)

            wait_gather(gbuf_b, gsem_b)

            @pl.when(m >= 1)
            def _():
                wait_writes(pbuf_b, wsem_b)

            select(kb, gbuf_b, pbuf_b)
            start_writes(kb, pbuf_b, wsem_b)

            @pl.when(kb + 2 < _NCHUNK)
            def _():
                start_gather(kb + 2, gbuf_b, gsem_b)

            return 0

        lax.fori_loop(0, _NCHUNK // 2, pair, 0)
        wait_writes(pbuf_a, wsem_a)
        wait_writes(pbuf_b, wsem_b)

    return gather_kernel


_transpose = _make_transpose()
_gather = _make_gather()


@jax.jit
def kernel(token_ids, weight):
    flat_ids = token_ids.reshape(_N).astype(jnp.int32)
    shift = _TB.bit_length() - 1
    blk = flat_ids >> shift
    off = flat_ids & (_TB - 1)
    rows = (blk << (shift - 1)) | (off & (_TB // 2 - 1))
    halves = off >> (shift - 1)
    table128 = _transpose(weight.T)
    return _gather(rows, halves, table128)
